# baseline (device time: 37759 ns/iter reference)
import jax
import jax.numpy as jnp
from jax import lax
from jax.experimental import pallas as pl
from jax.experimental.pallas import tpu as pltpu

N_DEV = 4
B = 2
SQ = 256
SKV = 512
D = 768
DH2 = D // 2
H_LOC = 8
DH = 64
HD_LOC = H_LOC * DH
ROWS = B * SQ


def _body(x_ref, wq_ref, wo_ref, k_ref, v_ref, out_ref,
          attn_ref, cw_ref, ccw_ref,
          cw_send, cw_recv, ccw_send, ccw_recv):
    my = lax.axis_index("i")
    left = lax.rem(my - 1 + N_DEV, N_DEV)
    right = lax.rem(my + 1, N_DEV)

    barrier = pltpu.get_barrier_semaphore()
    for nbr in (left, right):
        pl.semaphore_signal(barrier, inc=1, device_id=(nbr,),
                            device_id_type=pl.DeviceIdType.MESH)
    pl.semaphore_wait(barrier, 2)

    q_all = jnp.dot(x_ref[...], wq_ref[...],
                    preferred_element_type=jnp.float32).astype(jnp.bfloat16)

    for bh in range(B * H_LOC):
        b, h = divmod(bh, H_LOC)
        q = q_all[b * SQ:(b + 1) * SQ, h * DH:(h + 1) * DH]
        k = k_ref[bh]
        v = v_ref[bh]
        s = lax.dot_general(q, k, (((1,), (1,)), ((), ())),
                            preferred_element_type=jnp.float32) * 0.125
        m = jnp.max(s, axis=1, keepdims=True)
        p = jnp.exp(s - m)
        l = jnp.sum(p, axis=1, keepdims=True)
        p = (p / l).astype(jnp.bfloat16)
        o = jnp.dot(p, v, preferred_element_type=jnp.float32)
        attn_ref[b * SQ:(b + 1) * SQ, h * DH:(h + 1) * DH] = o.astype(jnp.bfloat16)

    def ring(comm_ref, send_sems, recv_sems, hop, tgt):
        return pltpu.make_async_remote_copy(
            src_ref=comm_ref.at[hop],
            dst_ref=comm_ref.at[hop + 1],
            send_sem=send_sems.at[hop],
            recv_sem=recv_sems.at[hop],
            device_id=(tgt,),
            device_id_type=pl.DeviceIdType.MESH,
        )

    attn = attn_ref[...]
    wo = wo_ref[...]

    p_cw = jnp.dot(attn, wo[:, :DH2], preferred_element_type=jnp.float32)
    cw_ref[0] = p_cw.astype(jnp.bfloat16)
    r_cw = ring(cw_ref, cw_send, cw_recv, 0, right)
    r_cw.start()

    p_ccw = jnp.dot(attn, wo[:, DH2:], preferred_element_type=jnp.float32)
    ccw_ref[0] = p_ccw.astype(jnp.bfloat16)
    r_ccw = ring(ccw_ref, ccw_send, ccw_recv, 0, left)
    r_ccw.start()

    acc_cw = p_cw
    acc_ccw = p_ccw
    rdmas = [r_cw, r_ccw]
    for hop in range(N_DEV - 1):
        r_cw.wait_recv()
        if hop < N_DEV - 2:
            r_cw = ring(cw_ref, cw_send, cw_recv, hop + 1, right)
            r_cw.start()
            rdmas.append(r_cw)
        acc_cw = acc_cw + cw_ref[hop + 1].astype(jnp.float32)

        r_ccw.wait_recv()
        if hop < N_DEV - 2:
            r_ccw = ring(ccw_ref, ccw_send, ccw_recv, hop + 1, left)
            r_ccw.start()
            rdmas.append(r_ccw)
        acc_ccw = acc_ccw + ccw_ref[hop + 1].astype(jnp.float32)

    out_ref[:, :, :DH2] = acc_cw.reshape(B, SQ, DH2)
    out_ref[:, :, DH2:] = acc_ccw.reshape(B, SQ, DH2)

    for r in rdmas:
        r.wait_send()


def kernel(x, Wq, Wo, K_ext, V_ext):
    idx = lax.axis_index("i")
    xb = x.reshape(ROWS, D).astype(jnp.bfloat16)
    wqb = Wq.astype(jnp.bfloat16)
    wob = Wo.astype(jnp.bfloat16)
    k = lax.dynamic_slice_in_dim(K_ext, idx * H_LOC, H_LOC, axis=2)
    v = lax.dynamic_slice_in_dim(V_ext, idx * H_LOC, H_LOC, axis=2)
    kb = k.transpose(0, 2, 1, 3).reshape(B * H_LOC, SKV, DH).astype(jnp.bfloat16)
    vb = v.transpose(0, 2, 1, 3).reshape(B * H_LOC, SKV, DH).astype(jnp.bfloat16)

    out = pl.pallas_call(
        _body,
        out_shape=jax.ShapeDtypeStruct((B, SQ, D), jnp.float32),
        in_specs=[pl.BlockSpec(memory_space=pltpu.VMEM)] * 5,
        out_specs=pl.BlockSpec(memory_space=pltpu.VMEM),
        scratch_shapes=[
            pltpu.VMEM((ROWS, HD_LOC), jnp.bfloat16),
            pltpu.VMEM((N_DEV, ROWS, DH2), jnp.bfloat16),
            pltpu.VMEM((N_DEV, ROWS, DH2), jnp.bfloat16),
            pltpu.SemaphoreType.DMA((N_DEV - 1,)),
            pltpu.SemaphoreType.DMA((N_DEV - 1,)),
            pltpu.SemaphoreType.DMA((N_DEV - 1,)),
            pltpu.SemaphoreType.DMA((N_DEV - 1,)),
        ],
        compiler_params=pltpu.CompilerParams(collective_id=0),
    )(xb, wqb, wob, kb, vb)
    return out


# device time: 30496 ns/iter; 1.2382x vs baseline; 1.2382x over previous
import jax
import jax.numpy as jnp
from jax import lax
from jax.experimental import pallas as pl
from jax.experimental.pallas import tpu as pltpu

N_DEV = 4
B = 2
SQ = 256
SKV = 512
D = 768
DHALF = D // 2
H_LOC = 8
DH = 64
HD_LOC = H_LOC * DH
ROWS = B * SQ

_SCHEDULE = ((0, 0), (0, 1), (1, 0), (0, 2), (1, 1), (1, 2))


def _body(x_ref, wq_ref, wo_ref, k_ref, v_ref, out_ref,
          cw_ref, ccw_ref, cw_send, cw_recv, ccw_send, ccw_recv):
    my = lax.axis_index("i")
    left = lax.rem(my - 1 + N_DEV, N_DEV)
    right = lax.rem(my + 1, N_DEV)

    barrier = pltpu.get_barrier_semaphore()
    for nbr in (left, right):
        pl.semaphore_signal(barrier, inc=1, device_id=(nbr,),
                            device_id_type=pl.DeviceIdType.MESH)
    pl.semaphore_wait(barrier, 2)

    wq = wq_ref[...].astype(jnp.bfloat16)
    wo = wo_ref[...].astype(jnp.bfloat16)

    def make(ring_ref, sends, recvs, c, hop, tgt):
        return pltpu.make_async_remote_copy(
            src_ref=ring_ref.at[c, hop],
            dst_ref=ring_ref.at[c, hop + 1],
            send_sem=sends.at[c, hop],
            recv_sem=recvs.at[c, hop],
            device_id=(tgt,),
            device_id_type=pl.DeviceIdType.MESH,
        )

    cw_r = [[None] * (N_DEV - 1) for _ in range(B)]
    ccw_r = [[None] * (N_DEV - 1) for _ in range(B)]
    acc_cw = [None] * B
    acc_ccw = [None] * B

    for c in range(B):
        xc = x_ref[c * SQ:(c + 1) * SQ, :].astype(jnp.bfloat16)
        q = (jnp.dot(xc, wq, preferred_element_type=jnp.float32)
             * 0.125).astype(jnp.bfloat16)

        outs = []
        for h in range(H_LOC):
            qh = q[:, h * DH:(h + 1) * DH]
            kh = k_ref[c, :, h * DH:(h + 1) * DH]
            vh = v_ref[c, :, h * DH:(h + 1) * DH]
            s = lax.dot_general(qh, kh, (((1,), (1,)), ((), ())),
                                preferred_element_type=jnp.float32)
            p = jnp.exp(s)
            l = jnp.sum(p, axis=1, keepdims=True)
            o = jnp.dot(p.astype(jnp.bfloat16), vh,
                        preferred_element_type=jnp.float32)
            outs.append((o * (1.0 / l)).astype(jnp.bfloat16))
        attn = jnp.concatenate(outs, axis=1)

        p_cw = jnp.dot(attn, wo[:, :DHALF],
                       preferred_element_type=jnp.float32)
        cw_ref[c, 0] = p_cw.astype(jnp.bfloat16)
        cw_r[c][0] = make(cw_ref, cw_send, cw_recv, c, 0, right)
        cw_r[c][0].start()

        p_ccw = jnp.dot(attn, wo[:, DHALF:],
                        preferred_element_type=jnp.float32)
        ccw_ref[c, 0] = p_ccw.astype(jnp.bfloat16)
        ccw_r[c][0] = make(ccw_ref, ccw_send, ccw_recv, c, 0, left)
        ccw_r[c][0].start()

        acc_cw[c] = p_cw
        acc_ccw[c] = p_ccw

    for (c, h) in _SCHEDULE:
        cw_r[c][h].wait_recv()
        if h < N_DEV - 2:
            cw_r[c][h + 1] = make(cw_ref, cw_send, cw_recv, c, h + 1, right)
            cw_r[c][h + 1].start()
        acc_cw[c] = acc_cw[c] + cw_ref[c, h + 1].astype(jnp.float32)

        ccw_r[c][h].wait_recv()
        if h < N_DEV - 2:
            ccw_r[c][h + 1] = make(ccw_ref, ccw_send, ccw_recv, c, h + 1, left)
            ccw_r[c][h + 1].start()
        acc_ccw[c] = acc_ccw[c] + ccw_ref[c, h + 1].astype(jnp.float32)

    for c in range(B):
        out_ref[c, :, :DHALF] = acc_cw[c]
        out_ref[c, :, DHALF:] = acc_ccw[c]

    for c in range(B):
        for h in range(N_DEV - 1):
            cw_r[c][h].wait_send()
            ccw_r[c][h].wait_send()


def kernel(x, Wq, Wo, K_ext, V_ext):
    idx = lax.axis_index("i")
    x2 = x.reshape(ROWS, D)
    k = lax.dynamic_slice_in_dim(K_ext, idx * H_LOC, H_LOC, axis=2)
    v = lax.dynamic_slice_in_dim(V_ext, idx * H_LOC, H_LOC, axis=2)
    kb = k.reshape(B, SKV, HD_LOC).astype(jnp.bfloat16)
    vb = v.reshape(B, SKV, HD_LOC).astype(jnp.bfloat16)

    out = pl.pallas_call(
        _body,
        out_shape=jax.ShapeDtypeStruct((B, SQ, D), jnp.float32),
        in_specs=[pl.BlockSpec(memory_space=pltpu.VMEM)] * 5,
        out_specs=pl.BlockSpec(memory_space=pltpu.VMEM),
        scratch_shapes=[
            pltpu.VMEM((B, N_DEV, SQ, DHALF), jnp.bfloat16),
            pltpu.VMEM((B, N_DEV, SQ, DHALF), jnp.bfloat16),
            pltpu.SemaphoreType.DMA((B, N_DEV - 1)),
            pltpu.SemaphoreType.DMA((B, N_DEV - 1)),
            pltpu.SemaphoreType.DMA((B, N_DEV - 1)),
            pltpu.SemaphoreType.DMA((B, N_DEV - 1)),
        ],
        compiler_params=pltpu.CompilerParams(collective_id=0),
    )(x2, Wq, Wo, kb, vb)
    return out


# device time: 26239 ns/iter; 1.4390x vs baseline; 1.1622x over previous
import jax
import jax.numpy as jnp
from jax import lax
from jax.experimental import pallas as pl
from jax.experimental.pallas import tpu as pltpu

N_DEV = 4
B = 2
SQ = 256
SKV = 512
D = 768
H_LOC = 8
DH = 64
HD_LOC = H_LOC * DH
ROWS = B * SQ
CH = 4
CROWS = ROWS // CH


def _body(x_ref, wq_ref, wo_ref, k_ref, v_ref, out_ref,
          sbuf, rbuf, ssem, rsem):
    my = lax.axis_index("i")
    px = 3 - my
    py = jnp.bitwise_xor(my, 1)

    barrier = pltpu.get_barrier_semaphore()
    for nbr in (px, py):
        pl.semaphore_signal(barrier, inc=1, device_id=(nbr,),
                            device_id_type=pl.DeviceIdType.MESH)

    wq = wq_ref[...].astype(jnp.bfloat16)
    wo = wo_ref[...].astype(jnp.bfloat16)

    def exch(ph, c, tgt):
        return pltpu.make_async_remote_copy(
            src_ref=sbuf.at[ph, c],
            dst_ref=rbuf.at[ph, c],
            send_sem=ssem.at[ph, c],
            recv_sem=rsem.at[ph, c],
            device_id=(tgt,),
            device_id_type=pl.DeviceIdType.MESH,
        )

    ex = [[None] * CH, [None] * CH]
    part = [None] * CH
    sum_x = [None] * CH

    def compute_chunk(c):
        b = c // (CH // B)
        xc = x_ref[c * CROWS:(c + 1) * CROWS, :].astype(jnp.bfloat16)
        q = (jnp.dot(xc, wq, preferred_element_type=jnp.float32)
             * 0.125).astype(jnp.bfloat16)
        outs = []
        for h in range(H_LOC):
            qh = q[:, h * DH:(h + 1) * DH]
            kh = k_ref[b, :, h * DH:(h + 1) * DH]
            vh = v_ref[b, :, h * DH:(h + 1) * DH]
            s = lax.dot_general(qh, kh, (((1,), (1,)), ((), ())),
                                preferred_element_type=jnp.float32)
            p = jnp.exp(s)
            l = jnp.sum(p, axis=1, keepdims=True)
            o = jnp.dot(p.astype(jnp.bfloat16), vh,
                        preferred_element_type=jnp.float32)
            outs.append((o * (1.0 / l)).astype(jnp.bfloat16))
        attn = jnp.concatenate(outs, axis=1)
        part[c] = jnp.dot(attn, wo,
                          preferred_element_type=jnp.float32)
        sbuf[0, c] = part[c].astype(jnp.bfloat16)

    def start_x(c):
        ex[0][c] = exch(0, c, px)
        ex[0][c].start()

    def finish_x_start_y(c):
        ex[0][c].wait_recv()
        sum_x[c] = part[c] + rbuf[0, c].astype(jnp.float32)
        sbuf[1, c] = sum_x[c].astype(jnp.bfloat16)
        ex[1][c] = exch(1, c, py)
        ex[1][c].start()

    def finish_y(c):
        ex[1][c].wait_recv()
        total = sum_x[c] + rbuf[1, c].astype(jnp.float32)
        b, r = c // (CH // B), (c % (CH // B)) * CROWS
        out_ref[b, r:r + CROWS, :] = total

    compute_chunk(0)
    pl.semaphore_wait(barrier, 2)
    start_x(0)
    compute_chunk(1)
    start_x(1)
    finish_x_start_y(0)
    compute_chunk(2)
    start_x(2)
    finish_x_start_y(1)
    compute_chunk(3)
    start_x(3)
    finish_x_start_y(2)
    finish_y(0)
    finish_x_start_y(3)
    finish_y(1)
    finish_y(2)
    finish_y(3)

    for ph in range(2):
        for c in range(CH):
            ex[ph][c].wait_send()


def kernel(x, Wq, Wo, K_ext, V_ext):
    idx = lax.axis_index("i")
    x2 = x.reshape(ROWS, D)
    k = lax.dynamic_slice_in_dim(K_ext, idx * H_LOC, H_LOC, axis=2)
    v = lax.dynamic_slice_in_dim(V_ext, idx * H_LOC, H_LOC, axis=2)
    kb = k.reshape(B, SKV, HD_LOC).astype(jnp.bfloat16)
    vb = v.reshape(B, SKV, HD_LOC).astype(jnp.bfloat16)

    out = pl.pallas_call(
        _body,
        out_shape=jax.ShapeDtypeStruct((B, SQ, D), jnp.float32),
        in_specs=[pl.BlockSpec(memory_space=pltpu.VMEM)] * 5,
        out_specs=pl.BlockSpec(memory_space=pltpu.VMEM),
        scratch_shapes=[
            pltpu.VMEM((2, CH, CROWS, D), jnp.bfloat16),
            pltpu.VMEM((2, CH, CROWS, D), jnp.bfloat16),
            pltpu.SemaphoreType.DMA((2, CH)),
            pltpu.SemaphoreType.DMA((2, CH)),
        ],
        compiler_params=pltpu.CompilerParams(collective_id=0),
    )(x2, Wq, Wo, kb, vb)
    return out


# device time: 25460 ns/iter; 1.4831x vs baseline; 1.0306x over previous
import jax
import jax.numpy as jnp
from jax import lax
from jax.experimental import pallas as pl
from jax.experimental.pallas import tpu as pltpu

N_DEV = 4
B = 2
SQ = 256
SKV = 512
D = 768
H_LOC = 8
DH = 64
HD_LOC = H_LOC * DH
ROWS = B * SQ
CH_SIZES = (160, 96, 160, 96)
CH_STARTS = (0, 160, 256, 416)
CH = len(CH_SIZES)
CMAX = max(CH_SIZES)


def _body(x_ref, wq_ref, wo_ref, k_ref, v_ref, out_ref,
          sbuf, rbuf, ssem, rsem):
    my = lax.axis_index("i")
    px = 3 - my
    py = jnp.bitwise_xor(my, 1)

    barrier = pltpu.get_barrier_semaphore()
    for nbr in (px, py):
        pl.semaphore_signal(barrier, inc=1, device_id=(nbr,),
                            device_id_type=pl.DeviceIdType.MESH)

    wq = wq_ref[...].astype(jnp.bfloat16)
    wo = wo_ref[...].astype(jnp.bfloat16)

    def exch(ph, c, tgt):
        n = CH_SIZES[c]
        return pltpu.make_async_remote_copy(
            src_ref=sbuf.at[ph, c, pl.ds(0, n)],
            dst_ref=rbuf.at[ph, c, pl.ds(0, n)],
            send_sem=ssem.at[ph, c],
            recv_sem=rsem.at[ph, c],
            device_id=(tgt,),
            device_id_type=pl.DeviceIdType.MESH,
        )

    ex = [[None] * CH, [None] * CH]
    part = [None] * CH
    sum_x = [None] * CH

    def compute_chunk(c):
        b = c // (CH // B)
        r0, n = CH_STARTS[c], CH_SIZES[c]
        xc = x_ref[r0:r0 + n, :].astype(jnp.bfloat16)
        q = (jnp.dot(xc, wq, preferred_element_type=jnp.float32)
             * 0.125).astype(jnp.bfloat16)
        outs = []
        for h in range(H_LOC):
            qh = q[:, h * DH:(h + 1) * DH]
            kh = k_ref[b, :, h * DH:(h + 1) * DH]
            vh = v_ref[b, :, h * DH:(h + 1) * DH]
            s = lax.dot_general(qh, kh, (((1,), (1,)), ((), ())),
                                preferred_element_type=jnp.float32)
            p = jnp.exp(s)
            l = jnp.sum(p, axis=1, keepdims=True)
            o = jnp.dot(p.astype(jnp.bfloat16), vh,
                        preferred_element_type=jnp.float32)
            outs.append((o * (1.0 / l)).astype(jnp.bfloat16))
        attn = jnp.concatenate(outs, axis=1)
        part[c] = jnp.dot(attn, wo,
                          preferred_element_type=jnp.float32)
        sbuf[0, c, :n] = part[c].astype(jnp.bfloat16)

    def start_x(c):
        ex[0][c] = exch(0, c, px)
        ex[0][c].start()

    def finish_x_start_y(c):
        n = CH_SIZES[c]
        ex[0][c].wait_recv()
        sum_x[c] = part[c] + rbuf[0, c, :n].astype(jnp.float32)
        sbuf[1, c, :n] = sum_x[c].astype(jnp.bfloat16)
        ex[1][c] = exch(1, c, py)
        ex[1][c].start()

    def finish_y(c):
        n = CH_SIZES[c]
        ex[1][c].wait_recv()
        total = sum_x[c] + rbuf[1, c, :n].astype(jnp.float32)
        b = c // (CH // B)
        r = CH_STARTS[c] - b * SQ
        out_ref[b, r:r + n, :] = total

    compute_chunk(0)
    pl.semaphore_wait(barrier, 2)
    start_x(0)
    compute_chunk(1)
    start_x(1)
    compute_chunk(2)
    start_x(2)
    finish_x_start_y(0)
    compute_chunk(3)
    start_x(3)
    finish_x_start_y(1)
    finish_x_start_y(2)
    finish_x_start_y(3)
    finish_y(0)
    finish_y(1)
    finish_y(2)
    finish_y(3)

    for ph in range(2):
        for c in range(CH):
            ex[ph][c].wait_send()


def kernel(x, Wq, Wo, K_ext, V_ext):
    idx = lax.axis_index("i")
    x2 = x.reshape(ROWS, D)
    k = lax.dynamic_slice_in_dim(K_ext, idx * H_LOC, H_LOC, axis=2)
    v = lax.dynamic_slice_in_dim(V_ext, idx * H_LOC, H_LOC, axis=2)
    kb = k.reshape(B, SKV, HD_LOC).astype(jnp.bfloat16)
    vb = v.reshape(B, SKV, HD_LOC).astype(jnp.bfloat16)

    out = pl.pallas_call(
        _body,
        out_shape=jax.ShapeDtypeStruct((B, SQ, D), jnp.float32),
        in_specs=[pl.BlockSpec(memory_space=pltpu.VMEM)] * 5,
        out_specs=pl.BlockSpec(memory_space=pltpu.VMEM),
        scratch_shapes=[
            pltpu.VMEM((2, CH, CMAX, D), jnp.bfloat16),
            pltpu.VMEM((2, CH, CMAX, D), jnp.bfloat16),
            pltpu.SemaphoreType.DMA((2, CH)),
            pltpu.SemaphoreType.DMA((2, CH)),
        ],
        compiler_params=pltpu.CompilerParams(collective_id=0),
    )(x2, Wq, Wo, kb, vb)
    return out
